# Initial kernel scaffold; baseline (speedup 1.0000x reference)
import jax
import jax.numpy as jnp
from jax.experimental import pallas as pl


def kernel(x, flow, sample_grid):
    raise NotImplementedError("write your pallas kernel here")


# speedup vs baseline: 1.6715x; 1.6715x over previous
import functools

import jax
import jax.numpy as jnp
from jax import lax
from jax.experimental import pallas as pl
from jax.experimental.pallas import tpu as pltpu
from jax.experimental.pallas import tpu_sc as plsc

D = H = W = 128
P = D * H * W
NB = 2
NP = NB * P
NWORK = 32
CHUNK = NP // NWORK
BLK = 256
NSUB = BLK // 128
NGRP = BLK // 16
NBLK = CHUNK // BLK


def _sc_body(x_hbm, fl_hbm, sg_hbm, out_hbm,
             gbuf_f, gbuf_s, idx_buf, t_buf, val_buf, out_buf,
             sem_in, sem_g, sem_out):
    cid = lax.axis_index("c")
    sid = lax.axis_index("s")
    wid = cid * 16 + sid
    base_pt = wid * CHUNK
    batch_off = cid * P

    def block(i, carry):
        pt0 = base_pt + i * BLK
        cps = [
            pltpu.async_copy(fl_hbm.at[pl.ds(c, 1), pl.ds(pt0, BLK)],
                             gbuf_f.at[pl.ds(c, 1), :], sem_in)
            for c in range(3)
        ] + [
            pltpu.async_copy(sg_hbm.at[pl.ds(c, 1), pl.ds(pt0, BLK)],
                             gbuf_s.at[pl.ds(c, 1), :], sem_in)
            for c in range(3)
        ]
        for c in cps:
            c.wait()

        for g in range(NGRP):
            j = g // 8
            col = (g % 8) * 16
            s = pl.ds(col, 16)
            sg = pl.ds(g * 16, 16)
            fx = gbuf_f[0, sg]
            fy = gbuf_f[1, sg]
            fz = gbuf_f[2, sg]
            sx = gbuf_s[0, sg]
            sy = gbuf_s[1, sg]
            sz = gbuf_s[2, sg]
            gx = (sx + fx + 1.0) * 0.5 * (W - 1.0)
            gy = (sy + fy + 1.0) * 0.5 * (H - 1.0)
            gz = (sz + fz + 1.0) * 0.5 * (D - 1.0)
            gx = jnp.clip(gx, 0.0, W - 1.0)
            gy = jnp.clip(gy, 0.0, H - 1.0)
            gz = jnp.clip(gz, 0.0, D - 1.0)
            ix0 = gx.astype(jnp.int32)
            iy0 = gy.astype(jnp.int32)
            iz0 = gz.astype(jnp.int32)
            tx = gx - ix0.astype(jnp.float32)
            ty = gy - iy0.astype(jnp.float32)
            tz = gz - iz0.astype(jnp.float32)
            ix1 = jnp.minimum(ix0 + 1, W - 1)
            iy1 = jnp.minimum(iy0 + 1, H - 1)
            iz1 = jnp.minimum(iz0 + 1, D - 1)

            a0 = (((iz0 << 7) + iy0) << 7) + batch_off
            a1 = (((iz0 << 7) + iy1) << 7) + batch_off
            b0 = (((iz1 << 7) + iy0) << 7) + batch_off
            b1 = (((iz1 << 7) + iy1) << 7) + batch_off
            corners = (a0 + ix0, a0 + ix1, a1 + ix0, a1 + ix1,
                       b0 + ix0, b0 + ix1, b1 + ix0, b1 + ix1)
            for k in range(8):
                idx_buf[k * NSUB + j, s] = corners[k]
            t_buf[0 * NSUB + j, s] = tx
            t_buf[1 * NSUB + j, s] = ty
            t_buf[2 * NSUB + j, s] = tz

        gcopies = [
            pltpu.async_copy(x_hbm.at[idx_buf.at[r]], val_buf.at[r], sem_g)
            for r in range(8 * NSUB)
        ]
        for c in gcopies:
            c.wait()

        for g in range(NGRP):
            j = g // 8
            col = (g % 8) * 16
            s = pl.ds(col, 16)
            tx = t_buf[0 * NSUB + j, s]
            ty = t_buf[1 * NSUB + j, s]
            tz = t_buf[2 * NSUB + j, s]
            ux = 1.0 - tx
            uy = 1.0 - ty
            uz = 1.0 - tz
            w00 = uz * uy
            w01 = uz * ty
            w10 = tz * uy
            w11 = tz * ty
            v = [val_buf[k * NSUB + j, s] for k in range(8)]
            r = ((v[0] * ux + v[1] * tx) * w00
                 + (v[2] * ux + v[3] * tx) * w01
                 + (v[4] * ux + v[5] * tx) * w10
                 + (v[6] * ux + v[7] * tx) * w11)
            out_buf[pl.ds(g * 16, 16)] = r

        pltpu.async_copy(out_buf, out_hbm.at[pl.ds(pt0, BLK)], sem_out).wait()
        return carry

    lax.fori_loop(0, NBLK, block, 0)


@functools.lru_cache(maxsize=1)
def _build():
    return pl.kernel(
        _sc_body,
        out_type=jax.ShapeDtypeStruct((NP,), jnp.float32),
        mesh=plsc.VectorSubcoreMesh(
            core_axis_name="c", subcore_axis_name="s",
            num_cores=2, num_subcores=16),
        scratch_types=[
            pltpu.VMEM((3, BLK), jnp.float32),
            pltpu.VMEM((3, BLK), jnp.float32),
            pltpu.VMEM((8 * NSUB, 128), jnp.int32),
            pltpu.VMEM((3 * NSUB, 128), jnp.float32),
            pltpu.VMEM((8 * NSUB, 128), jnp.float32),
            pltpu.VMEM((BLK,), jnp.float32),
            pltpu.SemaphoreType.DMA,
            pltpu.SemaphoreType.DMA,
            pltpu.SemaphoreType.DMA,
        ],
    )


def kernel(x, flow, sample_grid):
    fl = jnp.transpose(flow.reshape(NP, 3))
    sg = jnp.transpose(sample_grid.reshape(NP, 3))
    out = _build()(x.reshape(-1), fl, sg)
    return out.reshape(x.shape)
